# baseline (speedup 1.0000x reference)
import functools

import jax
import jax.numpy as jnp
from jax import lax
from jax.experimental import pallas as pl
from jax.experimental.pallas import tpu as pltpu
from jax.experimental.pallas import tpu_sc as plsc

EMB = 1024
HID = 1024
N_ROUTED = 6
TOP_K = 2
N_TOK = 2048
TILE = 256
N_CHUNK = N_TOK // TILE
T_MAX = 21
R_PAD = T_MAX * TILE
META_PAD = 64
NW = 32
TPW = N_TOK // NW


def _router_body(x_ref, gw_ref, gb_ref, bias_ref,
                 p0_ref, p1_ref, w0_ref, w1_ref, meta_ref):
    xb = x_ref[...]
    logits = jnp.dot(xb, gw_ref[...], preferred_element_type=jnp.float32)
    logits = logits + gb_ref[...]
    m = jnp.max(logits, axis=-1, keepdims=True)
    ex = jnp.exp(logits - m)
    probs = ex / jnp.sum(ex, axis=-1, keepdims=True)
    biased = probs + bias_ref[...]

    iota = lax.broadcasted_iota(jnp.int32, (N_TOK, N_ROUTED), 1)
    m1 = jnp.max(biased, axis=-1, keepdims=True)
    e0 = jnp.min(jnp.where(biased == m1, iota, N_ROUTED), axis=-1,
                 keepdims=True)
    p0w = jnp.sum(jnp.where(iota == e0, probs, 0.0), axis=-1, keepdims=True)
    biased2 = jnp.where(iota == e0, -1e30, biased)
    m2 = jnp.max(biased2, axis=-1, keepdims=True)
    e1 = jnp.min(jnp.where(biased2 == m2, iota, N_ROUTED), axis=-1,
                 keepdims=True)
    p1w = jnp.sum(jnp.where(iota == e1, probs, 0.0), axis=-1, keepdims=True)
    denom = p0w + p1w
    w0_ref[...] = p0w / denom
    w1_ref[...] = p1w / denom

    oh0 = (iota == e0).astype(jnp.float32)
    oh1 = (iota == e1).astype(jnp.float32)

    ri = lax.broadcasted_iota(jnp.int32, (TILE, TILE), 0)
    ci = lax.broadcasted_iota(jnp.int32, (TILE, TILE), 1)
    ltri = (ci < ri).astype(jnp.float32)

    def excl_cumsum(oh):
        off = jnp.zeros((1, N_ROUTED), jnp.float32)
        parts = []
        for c in range(N_CHUNK):
            xc = oh[c * TILE:(c + 1) * TILE, :]
            parts.append(jnp.dot(ltri, xc, preferred_element_type=jnp.float32)
                         + off)
            off = off + jnp.sum(xc, axis=0, keepdims=True)
        return jnp.concatenate(parts, axis=0), off

    rank0, cnt0 = excl_cumsum(oh0)
    rank1, cnt1 = excl_cumsum(oh1)
    rank1 = rank1 + cnt0
    counts = cnt0 + cnt1

    cnt_i = counts.astype(jnp.int32)
    nt = (cnt_i + (TILE - 1)) // TILE
    nt_f = nt.astype(jnp.float32)
    r6 = lax.broadcasted_iota(jnp.int32, (N_ROUTED, N_ROUTED), 0)
    c6 = lax.broadcasted_iota(jnp.int32, (N_ROUTED, N_ROUTED), 1)
    sutri = (r6 < c6).astype(jnp.float32)
    ts = jnp.dot(nt_f, sutri, preferred_element_type=jnp.float32)
    po = ts * float(TILE)

    p0_ref[...] = jnp.sum(oh0 * (po + rank0), axis=-1,
                          keepdims=True).astype(jnp.int32)
    p1_ref[...] = jnp.sum(oh1 * (po + rank1), axis=-1,
                          keepdims=True).astype(jnp.int32)

    total = jnp.sum(nt_f, axis=-1, keepdims=True)
    jt = lax.broadcasted_iota(jnp.int32, (META_PAD, N_ROUTED), 0
                              ).astype(jnp.float32)
    te = jnp.sum((ts <= jt).astype(jnp.float32), axis=-1, keepdims=True) - 1.0
    jt0 = lax.broadcasted_iota(jnp.int32, (META_PAD, 1), 0).astype(jnp.float32)
    te = jnp.where(jt0 < total, te, float(N_ROUTED))
    meta_ref[...] = te.astype(jnp.int32)


def _router(x, gate_w, gb, bias2):
    return pl.pallas_call(
        _router_body,
        out_shape=[
            jax.ShapeDtypeStruct((N_TOK, 1), jnp.int32),
            jax.ShapeDtypeStruct((N_TOK, 1), jnp.int32),
            jax.ShapeDtypeStruct((N_TOK, 1), jnp.float32),
            jax.ShapeDtypeStruct((N_TOK, 1), jnp.float32),
            jax.ShapeDtypeStruct((META_PAD, 1), jnp.int32),
        ],
    )(x, gate_w, gb, bias2)


def _sc_dispatch(x, p0, p1):
    @functools.partial(
        pl.kernel,
        mesh=plsc.VectorSubcoreMesh(core_axis_name="c", subcore_axis_name="s"),
        out_type=jax.ShapeDtypeStruct((R_PAD, EMB), jnp.float32),
        scratch_types=[
            pltpu.VMEM((TPW,), jnp.int32),
            pltpu.VMEM((TPW,), jnp.int32),
            pltpu.VMEM((TPW, EMB), jnp.float32),
            pltpu.SemaphoreType.DMA,
            pltpu.SemaphoreType.DMA,
        ],
    )
    def _body(x_hbm, p0_hbm, p1_hbm, xg_hbm, idx0_v, idx1_v, rows_v,
              sem0, sem1):
        wid = lax.axis_index("s") * 2 + lax.axis_index("c")
        base = wid * TPW
        pltpu.sync_copy(p0_hbm.at[pl.ds(base, TPW)], idx0_v)
        pltpu.sync_copy(p1_hbm.at[pl.ds(base, TPW)], idx1_v)
        pltpu.sync_copy(x_hbm.at[pl.ds(base, TPW)], rows_v)
        cp0 = pltpu.async_copy(rows_v, xg_hbm.at[idx0_v], sem0)
        cp1 = pltpu.async_copy(rows_v, xg_hbm.at[idx1_v], sem1)
        cp0.wait()
        cp1.wait()

    return _body(x, p0, p1)


def _sc_gather(y, p0, p1):
    @functools.partial(
        pl.kernel,
        mesh=plsc.VectorSubcoreMesh(core_axis_name="c", subcore_axis_name="s"),
        out_type=[
            jax.ShapeDtypeStruct((N_TOK, EMB), jnp.float32),
            jax.ShapeDtypeStruct((N_TOK, EMB), jnp.float32),
        ],
        scratch_types=[
            pltpu.VMEM((TPW,), jnp.int32),
            pltpu.VMEM((TPW,), jnp.int32),
            pltpu.VMEM((TPW, EMB), jnp.float32),
            pltpu.SemaphoreType.DMA,
        ],
    )
    def _body(y_hbm, p0_hbm, p1_hbm, y0_hbm, y1_hbm,
              idx0_v, idx1_v, rows_v, sem):
        wid = lax.axis_index("s") * 2 + lax.axis_index("c")
        base = wid * TPW
        pltpu.sync_copy(p0_hbm.at[pl.ds(base, TPW)], idx0_v)
        pltpu.sync_copy(p1_hbm.at[pl.ds(base, TPW)], idx1_v)
        pltpu.async_copy(y_hbm.at[idx0_v], rows_v, sem).wait()
        pltpu.sync_copy(rows_v, y0_hbm.at[pl.ds(base, TPW)])
        pltpu.async_copy(y_hbm.at[idx1_v], rows_v, sem).wait()
        pltpu.sync_copy(rows_v, y1_hbm.at[pl.ds(base, TPW)])

    return _body(y, p0, p1)


def _ffn_body(meta_ref, xg_ref, w1_ref, wg_ref, w2_ref, y_ref,
              w1b_ref, wgb_ref, w2b_ref, last_ref):
    t = pl.program_id(0)
    e = meta_ref[t]

    @pl.when((e < N_ROUTED) & ((t == 0) | (e != last_ref[0])))
    def _cast():
        w1b_ref[...] = w1_ref[0].astype(jnp.bfloat16)
        wgb_ref[...] = wg_ref[0].astype(jnp.bfloat16)
        w2b_ref[...] = w2_ref[0].astype(jnp.bfloat16)
        last_ref[0] = e

    @pl.when(e < N_ROUTED)
    def _compute():
        xb = xg_ref[...].astype(jnp.bfloat16)
        x1 = jnp.dot(xb, w1b_ref[...], preferred_element_type=jnp.float32)
        x1 = x1 * jax.nn.sigmoid(x1)
        x2 = jnp.dot(xb, wgb_ref[...], preferred_element_type=jnp.float32)
        h = (x1 * x2).astype(jnp.bfloat16)
        y_ref[...] = jnp.dot(h, w2b_ref[...],
                             preferred_element_type=jnp.float32)


def _ffn(meta, xg, rw1, rwg, rw2):
    def wsel(t, meta_ref):
        return (jnp.minimum(meta_ref[t], N_ROUTED - 1), 0, 0)

    return pl.pallas_call(
        _ffn_body,
        grid_spec=pltpu.PrefetchScalarGridSpec(
            num_scalar_prefetch=1,
            grid=(T_MAX,),
            in_specs=[
                pl.BlockSpec((TILE, EMB), lambda t, m: (t, 0)),
                pl.BlockSpec((1, EMB, HID), wsel),
                pl.BlockSpec((1, EMB, HID), wsel),
                pl.BlockSpec((1, HID, EMB), wsel),
            ],
            out_specs=pl.BlockSpec((TILE, EMB), lambda t, m: (t, 0)),
            scratch_shapes=[
                pltpu.VMEM((EMB, HID), jnp.bfloat16),
                pltpu.VMEM((EMB, HID), jnp.bfloat16),
                pltpu.VMEM((HID, EMB), jnp.bfloat16),
                pltpu.SMEM((1,), jnp.int32),
            ],
        ),
        out_shape=jax.ShapeDtypeStruct((R_PAD, EMB), jnp.float32),
    )(meta, xg, rw1, rwg, rw2)


def _shared_body(x_ref, w1s_ref, wgs_ref, w2s_ref, out_ref,
                 w1sb_ref, wgsb_ref, w2sb_ref):
    @pl.when(pl.program_id(0) == 0)
    def _cast():
        w1sb_ref[...] = w1s_ref[...].astype(jnp.bfloat16)
        wgsb_ref[...] = wgs_ref[...].astype(jnp.bfloat16)
        w2sb_ref[...] = w2s_ref[...].astype(jnp.bfloat16)

    xb = x_ref[...].astype(jnp.bfloat16)
    x1 = jnp.dot(xb, w1sb_ref[...], preferred_element_type=jnp.float32)
    x1 = x1 * jax.nn.sigmoid(x1)
    x2 = jnp.dot(xb, wgsb_ref[...], preferred_element_type=jnp.float32)
    h = (x1 * x2).astype(jnp.bfloat16)
    out_ref[...] = jnp.dot(h, w2sb_ref[...],
                           preferred_element_type=jnp.float32)


def _shared_ffn(x, w1s, wgs, w2s):
    nt = N_TOK // TILE
    return pl.pallas_call(
        _shared_body,
        grid=(nt,),
        in_specs=[
            pl.BlockSpec((TILE, EMB), lambda t: (t, 0)),
            pl.BlockSpec((EMB, 2 * HID), lambda t: (0, 0)),
            pl.BlockSpec((EMB, 2 * HID), lambda t: (0, 0)),
            pl.BlockSpec((2 * HID, EMB), lambda t: (0, 0)),
        ],
        out_specs=pl.BlockSpec((TILE, EMB), lambda t: (t, 0)),
        out_shape=jax.ShapeDtypeStruct((N_TOK, EMB), jnp.float32),
        scratch_shapes=[
            pltpu.VMEM((EMB, 2 * HID), jnp.bfloat16),
            pltpu.VMEM((EMB, 2 * HID), jnp.bfloat16),
            pltpu.VMEM((2 * HID, EMB), jnp.bfloat16),
        ],
    )(x, w1s, wgs, w2s)


def _combine_body(sh_ref, y0_ref, y1_ref, w0_ref, w1c_ref, out_ref):
    out_ref[...] = (sh_ref[...] + w0_ref[...] * y0_ref[...]
                    + w1c_ref[...] * y1_ref[...])


def _combine(sh, y0, y1, w0, w1c):
    nt = N_TOK // TILE
    return pl.pallas_call(
        _combine_body,
        grid=(nt,),
        in_specs=[
            pl.BlockSpec((TILE, EMB), lambda t: (t, 0)),
            pl.BlockSpec((TILE, EMB), lambda t: (t, 0)),
            pl.BlockSpec((TILE, EMB), lambda t: (t, 0)),
            pl.BlockSpec((TILE, 1), lambda t: (t, 0)),
            pl.BlockSpec((TILE, 1), lambda t: (t, 0)),
        ],
        out_specs=pl.BlockSpec((TILE, EMB), lambda t: (t, 0)),
        out_shape=jax.ShapeDtypeStruct((N_TOK, EMB), jnp.float32),
    )(sh, y0, y1, w0, w1c)


def kernel(x, routed_w1, routed_wg, routed_w2, shared_w1, shared_wg,
           shared_w2, gate_w, gate_b, biases):
    gb = gate_b.reshape(1, N_ROUTED)
    bias2 = biases.reshape(1, N_ROUTED)
    w1s = jnp.concatenate([shared_w1[0], shared_w1[1]], axis=1)
    wgs = jnp.concatenate([shared_wg[0], shared_wg[1]], axis=1)
    w2s = jnp.concatenate([shared_w2[0], shared_w2[1]], axis=0)

    p0c, p1c, w0c, w1c, metac = _router(x, gate_w, gb, bias2)
    p0 = p0c.reshape(N_TOK)
    p1 = p1c.reshape(N_TOK)
    meta = metac.reshape(META_PAD)

    xg = _sc_dispatch(x, p0, p1)
    y = _ffn(meta, xg, routed_w1, routed_wg, routed_w2)
    y0, y1 = _sc_gather(y, p0, p1)
    sh = _shared_ffn(x, w1s, wgs, w2s)
    return _combine(sh, y0, y1, w0c, w1c)


# speedup vs baseline: 1.1600x; 1.1600x over previous
import functools

import jax
import jax.numpy as jnp
from jax import lax
from jax.experimental import pallas as pl
from jax.experimental.pallas import tpu as pltpu
from jax.experimental.pallas import tpu_sc as plsc

EMB = 1024
HID = 1024
N_ROUTED = 6
TOP_K = 2
N_TOK = 2048
TILE = 256
N_CHUNK = N_TOK // TILE
T_MAX = 21
R_PAD = T_MAX * TILE
META_PAD = 64
NW = 32
TPW = N_TOK // NW


def _router_body(x_ref, gw_ref, gb_ref, bias_ref,
                 p0_ref, p1_ref, w0_ref, w1_ref, meta_ref):
    xb = x_ref[...]
    logits = jnp.dot(xb, gw_ref[...], preferred_element_type=jnp.float32)
    logits = logits + gb_ref[...]
    m = jnp.max(logits, axis=-1, keepdims=True)
    ex = jnp.exp(logits - m)
    probs = ex / jnp.sum(ex, axis=-1, keepdims=True)
    biased = probs + bias_ref[...]

    iota = lax.broadcasted_iota(jnp.int32, (N_TOK, N_ROUTED), 1)
    m1 = jnp.max(biased, axis=-1, keepdims=True)
    e0 = jnp.min(jnp.where(biased == m1, iota, N_ROUTED), axis=-1,
                 keepdims=True)
    p0w = jnp.sum(jnp.where(iota == e0, probs, 0.0), axis=-1, keepdims=True)
    biased2 = jnp.where(iota == e0, -1e30, biased)
    m2 = jnp.max(biased2, axis=-1, keepdims=True)
    e1 = jnp.min(jnp.where(biased2 == m2, iota, N_ROUTED), axis=-1,
                 keepdims=True)
    p1w = jnp.sum(jnp.where(iota == e1, probs, 0.0), axis=-1, keepdims=True)
    denom = p0w + p1w
    w0_ref[...] = p0w / denom
    w1_ref[...] = p1w / denom

    oh0 = (iota == e0).astype(jnp.float32)
    oh1 = (iota == e1).astype(jnp.float32)

    ri = lax.broadcasted_iota(jnp.int32, (TILE, TILE), 0)
    ci = lax.broadcasted_iota(jnp.int32, (TILE, TILE), 1)
    ltri = (ci < ri).astype(jnp.float32)

    def excl_cumsum(oh):
        off = jnp.zeros((1, N_ROUTED), jnp.float32)
        parts = []
        for c in range(N_CHUNK):
            xc = oh[c * TILE:(c + 1) * TILE, :]
            parts.append(jnp.dot(ltri, xc, preferred_element_type=jnp.float32)
                         + off)
            off = off + jnp.sum(xc, axis=0, keepdims=True)
        return jnp.concatenate(parts, axis=0), off

    rank0, cnt0 = excl_cumsum(oh0)
    rank1, cnt1 = excl_cumsum(oh1)
    rank1 = rank1 + cnt0
    counts = cnt0 + cnt1

    cnt_i = counts.astype(jnp.int32)
    nt = (cnt_i + (TILE - 1)) // TILE
    nt_f = nt.astype(jnp.float32)
    r6 = lax.broadcasted_iota(jnp.int32, (N_ROUTED, N_ROUTED), 0)
    c6 = lax.broadcasted_iota(jnp.int32, (N_ROUTED, N_ROUTED), 1)
    sutri = (r6 < c6).astype(jnp.float32)
    ts = jnp.dot(nt_f, sutri, preferred_element_type=jnp.float32)
    po = ts * float(TILE)

    p0_ref[...] = jnp.sum(oh0 * (po + rank0), axis=-1,
                          keepdims=True).astype(jnp.int32)
    p1_ref[...] = jnp.sum(oh1 * (po + rank1), axis=-1,
                          keepdims=True).astype(jnp.int32)

    total = jnp.sum(nt_f, axis=-1, keepdims=True)
    jt = lax.broadcasted_iota(jnp.int32, (META_PAD, N_ROUTED), 0
                              ).astype(jnp.float32)
    te = jnp.sum((ts <= jt).astype(jnp.float32), axis=-1, keepdims=True) - 1.0
    jt0 = lax.broadcasted_iota(jnp.int32, (META_PAD, 1), 0).astype(jnp.float32)
    te = jnp.where(jt0 < total, te, float(N_ROUTED))
    meta_ref[...] = te.astype(jnp.int32)


def _router(x, gate_w, gb, bias2):
    return pl.pallas_call(
        _router_body,
        out_shape=[
            jax.ShapeDtypeStruct((N_TOK, 1), jnp.int32),
            jax.ShapeDtypeStruct((N_TOK, 1), jnp.int32),
            jax.ShapeDtypeStruct((N_TOK, 1), jnp.float32),
            jax.ShapeDtypeStruct((N_TOK, 1), jnp.float32),
            jax.ShapeDtypeStruct((META_PAD, 1), jnp.int32),
        ],
    )(x, gate_w, gb, bias2)


def _sc_dispatch(x, p0, p1):
    @functools.partial(
        pl.kernel,
        mesh=plsc.VectorSubcoreMesh(core_axis_name="c", subcore_axis_name="s"),
        out_type=jax.ShapeDtypeStruct((R_PAD, EMB), jnp.float32),
        scratch_types=[
            pltpu.VMEM((TPW,), jnp.int32),
            pltpu.VMEM((TPW,), jnp.int32),
            pltpu.VMEM((TPW, EMB), jnp.float32),
            pltpu.SemaphoreType.DMA,
            pltpu.SemaphoreType.DMA,
        ],
    )
    def _body(x_hbm, p0_hbm, p1_hbm, xg_hbm, idx0_v, idx1_v, rows_v,
              sem0, sem1):
        wid = lax.axis_index("s") * 2 + lax.axis_index("c")
        base = wid * TPW
        pltpu.sync_copy(p0_hbm.at[pl.ds(base, TPW)], idx0_v)
        pltpu.sync_copy(p1_hbm.at[pl.ds(base, TPW)], idx1_v)
        pltpu.sync_copy(x_hbm.at[pl.ds(base, TPW)], rows_v)
        cp0 = pltpu.async_copy(rows_v, xg_hbm.at[idx0_v], sem0)
        cp1 = pltpu.async_copy(rows_v, xg_hbm.at[idx1_v], sem1)
        cp0.wait()
        cp1.wait()

    return _body(x, p0, p1)


def _sc_gather(y, p0, p1):
    @functools.partial(
        pl.kernel,
        mesh=plsc.VectorSubcoreMesh(core_axis_name="c", subcore_axis_name="s"),
        out_type=[
            jax.ShapeDtypeStruct((N_TOK, EMB), jnp.float32),
            jax.ShapeDtypeStruct((N_TOK, EMB), jnp.float32),
        ],
        scratch_types=[
            pltpu.VMEM((TPW,), jnp.int32),
            pltpu.VMEM((TPW,), jnp.int32),
            pltpu.VMEM((TPW, EMB), jnp.float32),
            pltpu.SemaphoreType.DMA,
        ],
    )
    def _body(y_hbm, p0_hbm, p1_hbm, y0_hbm, y1_hbm,
              idx0_v, idx1_v, rows_v, sem):
        wid = lax.axis_index("s") * 2 + lax.axis_index("c")
        base = wid * TPW
        pltpu.sync_copy(p0_hbm.at[pl.ds(base, TPW)], idx0_v)
        pltpu.sync_copy(p1_hbm.at[pl.ds(base, TPW)], idx1_v)
        pltpu.async_copy(y_hbm.at[idx0_v], rows_v, sem).wait()
        pltpu.sync_copy(rows_v, y0_hbm.at[pl.ds(base, TPW)])
        pltpu.async_copy(y_hbm.at[idx1_v], rows_v, sem).wait()
        pltpu.sync_copy(rows_v, y1_hbm.at[pl.ds(base, TPW)])

    return _body(y, p0, p1)


def _ffn_body(meta_ref, xg_ref, w1_ref, wg_ref, w2_ref, y_ref,
              w1b_ref, wgb_ref, w2b_ref, last_ref):
    t = pl.program_id(0)
    e = meta_ref[t]

    @pl.when((e < N_ROUTED) & ((t == 0) | (e != last_ref[0])))
    def _cast():
        w1b_ref[...] = w1_ref[0].astype(jnp.bfloat16)
        wgb_ref[...] = wg_ref[0].astype(jnp.bfloat16)
        w2b_ref[...] = w2_ref[0].astype(jnp.bfloat16)
        last_ref[0] = e

    @pl.when(e < N_ROUTED)
    def _compute():
        xb = xg_ref[...].astype(jnp.bfloat16)
        x1 = jnp.dot(xb, w1b_ref[...], preferred_element_type=jnp.float32)
        x1 = x1 * jax.nn.sigmoid(x1)
        x2 = jnp.dot(xb, wgb_ref[...], preferred_element_type=jnp.float32)
        h = (x1 * x2).astype(jnp.bfloat16)
        y_ref[...] = jnp.dot(h, w2b_ref[...],
                             preferred_element_type=jnp.float32)


def _ffn(meta, xg, rw1, rwg, rw2):
    def wsel(t, meta_ref):
        return (jnp.minimum(meta_ref[t], N_ROUTED - 1), 0, 0)

    return pl.pallas_call(
        _ffn_body,
        grid_spec=pltpu.PrefetchScalarGridSpec(
            num_scalar_prefetch=1,
            grid=(T_MAX,),
            in_specs=[
                pl.BlockSpec((TILE, EMB), lambda t, m: (t, 0)),
                pl.BlockSpec((1, EMB, HID), wsel),
                pl.BlockSpec((1, EMB, HID), wsel),
                pl.BlockSpec((1, HID, EMB), wsel),
            ],
            out_specs=pl.BlockSpec((TILE, EMB), lambda t, m: (t, 0)),
            scratch_shapes=[
                pltpu.VMEM((EMB, HID), jnp.bfloat16),
                pltpu.VMEM((EMB, HID), jnp.bfloat16),
                pltpu.VMEM((HID, EMB), jnp.bfloat16),
                pltpu.SMEM((1,), jnp.int32),
            ],
        ),
        out_shape=jax.ShapeDtypeStruct((R_PAD, EMB), jnp.float32),
    )(meta, xg, rw1, rwg, rw2)


def _shared_body(x_ref, w1_ref, wg_ref, w2_ref, out_ref,
                 w1b_ref, wgb_ref, w2b_ref):
    @pl.when(pl.program_id(0) == 0)
    def _cast():
        w1b_ref[...] = w1_ref[0].astype(jnp.bfloat16)
        wgb_ref[...] = wg_ref[0].astype(jnp.bfloat16)
        w2b_ref[...] = w2_ref[0].astype(jnp.bfloat16)

    xb = x_ref[...].astype(jnp.bfloat16)
    x1 = jnp.dot(xb, w1b_ref[...], preferred_element_type=jnp.float32)
    x1 = x1 * jax.nn.sigmoid(x1)
    x2 = jnp.dot(xb, wgb_ref[...], preferred_element_type=jnp.float32)
    h = (x1 * x2).astype(jnp.bfloat16)
    out_ref[...] = jnp.dot(h, w2b_ref[...],
                           preferred_element_type=jnp.float32)


def _shared_ffn_one(x, sw1, swg, sw2, e):
    nt = N_TOK // TILE
    return pl.pallas_call(
        _shared_body,
        grid=(nt,),
        in_specs=[
            pl.BlockSpec((TILE, EMB), lambda t: (t, 0)),
            pl.BlockSpec((1, EMB, HID), lambda t: (e, 0, 0)),
            pl.BlockSpec((1, EMB, HID), lambda t: (e, 0, 0)),
            pl.BlockSpec((1, HID, EMB), lambda t: (e, 0, 0)),
        ],
        out_specs=pl.BlockSpec((TILE, EMB), lambda t: (t, 0)),
        out_shape=jax.ShapeDtypeStruct((N_TOK, EMB), jnp.float32),
        scratch_shapes=[
            pltpu.VMEM((EMB, HID), jnp.bfloat16),
            pltpu.VMEM((EMB, HID), jnp.bfloat16),
            pltpu.VMEM((HID, EMB), jnp.bfloat16),
        ],
    )(x, sw1, swg, sw2)


def _combine_body(sh0_ref, sh1_ref, y0_ref, y1_ref, w0_ref, w1c_ref,
                  out_ref):
    out_ref[...] = (sh0_ref[...] + sh1_ref[...] + w0_ref[...] * y0_ref[...]
                    + w1c_ref[...] * y1_ref[...])


def _combine(sh0, sh1, y0, y1, w0, w1c):
    nt = N_TOK // TILE
    return pl.pallas_call(
        _combine_body,
        grid=(nt,),
        in_specs=[
            pl.BlockSpec((TILE, EMB), lambda t: (t, 0)),
            pl.BlockSpec((TILE, EMB), lambda t: (t, 0)),
            pl.BlockSpec((TILE, EMB), lambda t: (t, 0)),
            pl.BlockSpec((TILE, EMB), lambda t: (t, 0)),
            pl.BlockSpec((TILE, 1), lambda t: (t, 0)),
            pl.BlockSpec((TILE, 1), lambda t: (t, 0)),
        ],
        out_specs=pl.BlockSpec((TILE, EMB), lambda t: (t, 0)),
        out_shape=jax.ShapeDtypeStruct((N_TOK, EMB), jnp.float32),
    )(sh0, sh1, y0, y1, w0, w1c)


def kernel(x, routed_w1, routed_wg, routed_w2, shared_w1, shared_wg,
           shared_w2, gate_w, gate_b, biases):
    gb = gate_b.reshape(1, N_ROUTED)
    bias2 = biases.reshape(1, N_ROUTED)

    p0c, p1c, w0c, w1c, metac = _router(x, gate_w, gb, bias2)
    p0 = p0c.reshape(N_TOK)
    p1 = p1c.reshape(N_TOK)
    meta = metac.reshape(META_PAD)

    xg = _sc_dispatch(x, p0, p1)
    sh0 = _shared_ffn_one(x, shared_w1, shared_wg, shared_w2, 0)
    y = _ffn(meta, xg, routed_w1, routed_wg, routed_w2)
    y0, y1 = _sc_gather(y, p0, p1)
    sh1 = _shared_ffn_one(x, shared_w1, shared_wg, shared_w2, 1)
    return _combine(sh0, sh1, y0, y1, w0c, w1c)


# speedup vs baseline: 1.1974x; 1.0323x over previous
import functools

import jax
import jax.numpy as jnp
from jax import lax
from jax.experimental import pallas as pl
from jax.experimental.pallas import tpu as pltpu
from jax.experimental.pallas import tpu_sc as plsc

EMB = 1024
HID = 1024
N_ROUTED = 6
TOP_K = 2
N_TOK = 2048
TILE = 256
N_CHUNK = N_TOK // TILE
T_MAX = 21
R_PAD = T_MAX * TILE
META_PAD = 64
NW = 32
TPW = N_TOK // NW


def _router_body(x_ref, gw_ref, gb_ref, bias_ref,
                 p0_ref, p1_ref, w0_ref, w1_ref, meta_ref):
    xb = x_ref[...]
    logits = jnp.dot(xb, gw_ref[...], preferred_element_type=jnp.float32)
    logits = logits + gb_ref[...]
    m = jnp.max(logits, axis=-1, keepdims=True)
    ex = jnp.exp(logits - m)
    probs = ex / jnp.sum(ex, axis=-1, keepdims=True)
    biased = probs + bias_ref[...]

    iota = lax.broadcasted_iota(jnp.int32, (N_TOK, N_ROUTED), 1)
    m1 = jnp.max(biased, axis=-1, keepdims=True)
    e0 = jnp.min(jnp.where(biased == m1, iota, N_ROUTED), axis=-1,
                 keepdims=True)
    p0w = jnp.sum(jnp.where(iota == e0, probs, 0.0), axis=-1, keepdims=True)
    biased2 = jnp.where(iota == e0, -1e30, biased)
    m2 = jnp.max(biased2, axis=-1, keepdims=True)
    e1 = jnp.min(jnp.where(biased2 == m2, iota, N_ROUTED), axis=-1,
                 keepdims=True)
    p1w = jnp.sum(jnp.where(iota == e1, probs, 0.0), axis=-1, keepdims=True)
    denom = p0w + p1w
    w0_ref[...] = p0w / denom
    w1_ref[...] = p1w / denom

    oh0 = (iota == e0).astype(jnp.float32)
    oh1 = (iota == e1).astype(jnp.float32)

    ri = lax.broadcasted_iota(jnp.int32, (TILE, TILE), 0)
    ci = lax.broadcasted_iota(jnp.int32, (TILE, TILE), 1)
    ltri = (ci < ri).astype(jnp.float32)

    def excl_cumsum(oh):
        off = jnp.zeros((1, N_ROUTED), jnp.float32)
        parts = []
        for c in range(N_CHUNK):
            xc = oh[c * TILE:(c + 1) * TILE, :]
            parts.append(jnp.dot(ltri, xc, preferred_element_type=jnp.float32)
                         + off)
            off = off + jnp.sum(xc, axis=0, keepdims=True)
        return jnp.concatenate(parts, axis=0), off

    rank0, cnt0 = excl_cumsum(oh0)
    rank1, cnt1 = excl_cumsum(oh1)
    rank1 = rank1 + cnt0
    counts = cnt0 + cnt1

    cnt_i = counts.astype(jnp.int32)
    nt = (cnt_i + (TILE - 1)) // TILE
    nt_f = nt.astype(jnp.float32)
    r6 = lax.broadcasted_iota(jnp.int32, (N_ROUTED, N_ROUTED), 0)
    c6 = lax.broadcasted_iota(jnp.int32, (N_ROUTED, N_ROUTED), 1)
    sutri = (r6 < c6).astype(jnp.float32)
    ts = jnp.dot(nt_f, sutri, preferred_element_type=jnp.float32)
    po = ts * float(TILE)

    p0_ref[...] = jnp.sum(oh0 * (po + rank0), axis=-1,
                          keepdims=True).astype(jnp.int32)
    p1_ref[...] = jnp.sum(oh1 * (po + rank1), axis=-1,
                          keepdims=True).astype(jnp.int32)

    total = jnp.sum(nt_f, axis=-1, keepdims=True)
    jt = lax.broadcasted_iota(jnp.int32, (META_PAD, N_ROUTED), 0
                              ).astype(jnp.float32)
    te = jnp.sum((ts <= jt).astype(jnp.float32), axis=-1, keepdims=True) - 1.0
    jt0 = lax.broadcasted_iota(jnp.int32, (META_PAD, 1), 0).astype(jnp.float32)
    te = jnp.where(jt0 < total, te, float(N_ROUTED))
    meta_ref[...] = te.astype(jnp.int32)


def _router(x, gate_w, gb, bias2):
    return pl.pallas_call(
        _router_body,
        out_shape=[
            jax.ShapeDtypeStruct((N_TOK, 1), jnp.int32),
            jax.ShapeDtypeStruct((N_TOK, 1), jnp.int32),
            jax.ShapeDtypeStruct((N_TOK, 1), jnp.float32),
            jax.ShapeDtypeStruct((N_TOK, 1), jnp.float32),
            jax.ShapeDtypeStruct((META_PAD, 1), jnp.int32),
        ],
    )(x, gate_w, gb, bias2)


def _sc_dispatch(x, p0, p1):
    @functools.partial(
        pl.kernel,
        mesh=plsc.VectorSubcoreMesh(core_axis_name="c", subcore_axis_name="s"),
        out_type=jax.ShapeDtypeStruct((R_PAD, EMB), jnp.float32),
        scratch_types=[
            pltpu.VMEM((TPW,), jnp.int32),
            pltpu.VMEM((TPW,), jnp.int32),
            pltpu.VMEM((TPW, EMB), jnp.float32),
            pltpu.SemaphoreType.DMA,
            pltpu.SemaphoreType.DMA,
        ],
    )
    def _body(x_hbm, p0_hbm, p1_hbm, xg_hbm, idx0_v, idx1_v, rows_v,
              sem0, sem1):
        wid = lax.axis_index("s") * 2 + lax.axis_index("c")
        base = wid * TPW
        pltpu.sync_copy(p0_hbm.at[pl.ds(base, TPW)], idx0_v)
        pltpu.sync_copy(p1_hbm.at[pl.ds(base, TPW)], idx1_v)
        pltpu.sync_copy(x_hbm.at[pl.ds(base, TPW)], rows_v)
        cp0 = pltpu.async_copy(rows_v, xg_hbm.at[idx0_v], sem0)
        cp1 = pltpu.async_copy(rows_v, xg_hbm.at[idx1_v], sem1)
        cp0.wait()
        cp1.wait()

    return _body(x, p0, p1)


def _sc_gather(y, p0, p1):
    @functools.partial(
        pl.kernel,
        mesh=plsc.VectorSubcoreMesh(core_axis_name="c", subcore_axis_name="s"),
        out_type=[
            jax.ShapeDtypeStruct((N_TOK, EMB), jnp.float32),
            jax.ShapeDtypeStruct((N_TOK, EMB), jnp.float32),
        ],
        scratch_types=[
            pltpu.VMEM((TPW,), jnp.int32),
            pltpu.VMEM((TPW,), jnp.int32),
            pltpu.VMEM((TPW, EMB), jnp.float32),
            pltpu.SemaphoreType.DMA,
        ],
    )
    def _body(y_hbm, p0_hbm, p1_hbm, y0_hbm, y1_hbm,
              idx0_v, idx1_v, rows_v, sem):
        wid = lax.axis_index("s") * 2 + lax.axis_index("c")
        base = wid * TPW
        pltpu.sync_copy(p0_hbm.at[pl.ds(base, TPW)], idx0_v)
        pltpu.sync_copy(p1_hbm.at[pl.ds(base, TPW)], idx1_v)
        pltpu.async_copy(y_hbm.at[idx0_v], rows_v, sem).wait()
        pltpu.sync_copy(rows_v, y0_hbm.at[pl.ds(base, TPW)])
        pltpu.async_copy(y_hbm.at[idx1_v], rows_v, sem).wait()
        pltpu.sync_copy(rows_v, y1_hbm.at[pl.ds(base, TPW)])

    return _body(y, p0, p1)


def _ffn_body(meta_ref, xg_ref, w1_ref, wg_ref, w2_ref, y_ref,
              w1b_ref, wgb_ref, w2b_ref, last_ref):
    t = pl.program_id(0)
    e = meta_ref[t]

    @pl.when((e < N_ROUTED) & ((t == 0) | (e != last_ref[0])))
    def _cast():
        w1b_ref[...] = w1_ref[0].astype(jnp.bfloat16)
        wgb_ref[...] = wg_ref[0].astype(jnp.bfloat16)
        w2b_ref[...] = w2_ref[0].astype(jnp.bfloat16)
        last_ref[0] = e

    @pl.when(e < N_ROUTED)
    def _compute():
        xb = xg_ref[...].astype(jnp.bfloat16)
        x1 = jnp.dot(xb, w1b_ref[...], preferred_element_type=jnp.float32)
        x1 = x1 * jax.nn.sigmoid(x1)
        x2 = jnp.dot(xb, wgb_ref[...], preferred_element_type=jnp.float32)
        h = (x1 * x2).astype(jnp.bfloat16)
        y_ref[...] = jnp.dot(h, w2b_ref[...],
                             preferred_element_type=jnp.float32)


def _ffn(meta, xg, rw1, rwg, rw2):
    def wsel(t, meta_ref):
        return (jnp.minimum(meta_ref[t], N_ROUTED - 1), 0, 0)

    return pl.pallas_call(
        _ffn_body,
        grid_spec=pltpu.PrefetchScalarGridSpec(
            num_scalar_prefetch=1,
            grid=(T_MAX,),
            in_specs=[
                pl.BlockSpec((TILE, EMB), lambda t, m: (t, 0)),
                pl.BlockSpec((1, EMB, HID), wsel),
                pl.BlockSpec((1, EMB, HID), wsel),
                pl.BlockSpec((1, HID, EMB), wsel),
            ],
            out_specs=pl.BlockSpec((TILE, EMB), lambda t, m: (t, 0)),
            scratch_shapes=[
                pltpu.VMEM((EMB, HID), jnp.bfloat16),
                pltpu.VMEM((EMB, HID), jnp.bfloat16),
                pltpu.VMEM((HID, EMB), jnp.bfloat16),
                pltpu.SMEM((1,), jnp.int32),
            ],
        ),
        out_shape=jax.ShapeDtypeStruct((R_PAD, EMB), jnp.float32),
    )(meta, xg, rw1, rwg, rw2)


def _shared_body(x_ref, w1_ref, wg_ref, w2_ref, out_ref,
                 w1b_ref, wgb_ref, w2b_ref):
    @pl.when(pl.program_id(0) == 0)
    def _cast():
        w1b_ref[...] = w1_ref[0].astype(jnp.bfloat16)
        wgb_ref[...] = wg_ref[0].astype(jnp.bfloat16)
        w2b_ref[...] = w2_ref[0].astype(jnp.bfloat16)

    xb = x_ref[...].astype(jnp.bfloat16)
    x1 = jnp.dot(xb, w1b_ref[...], preferred_element_type=jnp.float32)
    x1 = x1 * jax.nn.sigmoid(x1)
    x2 = jnp.dot(xb, wgb_ref[...], preferred_element_type=jnp.float32)
    h = (x1 * x2).astype(jnp.bfloat16)
    out_ref[...] = jnp.dot(h, w2b_ref[...],
                           preferred_element_type=jnp.float32)


def _shared_ffn_one(x, sw1, swg, sw2, e):
    nt = N_TOK // TILE
    return pl.pallas_call(
        _shared_body,
        grid=(nt,),
        in_specs=[
            pl.BlockSpec((TILE, EMB), lambda t: (t, 0)),
            pl.BlockSpec((1, EMB, HID), lambda t: (e, 0, 0)),
            pl.BlockSpec((1, EMB, HID), lambda t: (e, 0, 0)),
            pl.BlockSpec((1, HID, EMB), lambda t: (e, 0, 0)),
        ],
        out_specs=pl.BlockSpec((TILE, EMB), lambda t: (t, 0)),
        out_shape=jax.ShapeDtypeStruct((N_TOK, EMB), jnp.float32),
        scratch_shapes=[
            pltpu.VMEM((EMB, HID), jnp.bfloat16),
            pltpu.VMEM((EMB, HID), jnp.bfloat16),
            pltpu.VMEM((HID, EMB), jnp.bfloat16),
        ],
    )(x, sw1, swg, sw2)


def _combine_body(sh0_ref, sh1_ref, y0_ref, y1_ref, w0_ref, w1c_ref,
                  out_ref):
    out_ref[...] = (sh0_ref[...] + sh1_ref[...] + w0_ref[...] * y0_ref[...]
                    + w1c_ref[...] * y1_ref[...])


def _combine(sh0, sh1, y0, y1, w0, w1c):
    nt = N_TOK // TILE
    return pl.pallas_call(
        _combine_body,
        grid=(nt,),
        in_specs=[
            pl.BlockSpec((TILE, EMB), lambda t: (t, 0)),
            pl.BlockSpec((TILE, EMB), lambda t: (t, 0)),
            pl.BlockSpec((TILE, EMB), lambda t: (t, 0)),
            pl.BlockSpec((TILE, EMB), lambda t: (t, 0)),
            pl.BlockSpec((TILE, 1), lambda t: (t, 0)),
            pl.BlockSpec((TILE, 1), lambda t: (t, 0)),
        ],
        out_specs=pl.BlockSpec((TILE, EMB), lambda t: (t, 0)),
        out_shape=jax.ShapeDtypeStruct((N_TOK, EMB), jnp.float32),
    )(sh0, sh1, y0, y1, w0, w1c)


def kernel(x, routed_w1, routed_wg, routed_w2, shared_w1, shared_wg,
           shared_w2, gate_w, gate_b, biases):
    gb = gate_b.reshape(1, N_ROUTED)
    bias2 = biases.reshape(1, N_ROUTED)

    p0c, p1c, w0c, w1c, metac = _router(x, gate_w, gb, bias2)
    p0 = p0c.reshape(N_TOK)
    p1 = p1c.reshape(N_TOK)
    meta = metac.reshape(META_PAD)

    xg = _sc_dispatch(x, p0, p1)
    sh0 = _shared_ffn_one(x, shared_w1, shared_wg, shared_w2, 0)
    xg, sh0 = lax.optimization_barrier((xg, sh0))
    y = _ffn(meta, xg, routed_w1, routed_wg, routed_w2)
    y0, y1 = _sc_gather(y, p0, p1)
    sh1 = _shared_ffn_one(x, shared_w1, shared_wg, shared_w2, 1)
    return _combine(sh0, sh1, y0, y1, w0c, w1c)
